# baseline (speedup 1.0000x reference)
import functools

import jax
import jax.numpy as jnp
from jax.experimental import pallas as pl
from jax.experimental.pallas import tpu as pltpu

LANES = 128
RB = 8


def _smooth_l1(d):
    a = jnp.abs(d)
    return jnp.where(a < 1.0, 0.5 * d * d, a - 0.5)


def _body(n_batch, n_gt, gtab_ref, tab_ref, anc_ref, cls_ref, pq_ref, out_ref):
    j = pl.program_id(0)

    @pl.when(j == 0)
    def _():
        out_ref[...] = jnp.zeros_like(out_ref)

    x0 = anc_ref[0]
    y0 = anc_ref[1]
    x1 = anc_ref[2]
    y1 = anc_ref[3]
    aw = x1 - x0
    ah = y1 - y0
    a1 = aw * ah
    iaw = 1.0 / aw
    iah = 1.0 / ah

    iota = jax.lax.broadcasted_iota(jnp.int16, (LANES, LANES), 0)
    onec = jnp.bfloat16(1.0)
    zeroc = jnp.bfloat16(0.0)

    for i in range(n_batch):
        base = i * n_gt

        def pair(g):
            ixmin = jnp.maximum(x0, gtab_ref[base + g, 0])
            iymin = jnp.maximum(y0, gtab_ref[base + g, 1])
            ixmax = jnp.minimum(x1, gtab_ref[base + g, 2])
            iymax = jnp.minimum(y1, gtab_ref[base + g, 3])
            dx = jnp.maximum(ixmax - ixmin, 0.0)
            dy = jnp.maximum(iymax - iymin, 0.0)
            inter = dx * dy
            union = (a1 + gtab_ref[base + g, 4]) - inter
            return inter, union

        bi, bu = pair(0)
        bidx = jnp.zeros_like(x0)

        def mstep(g, carry):
            bi, bu, bidx = carry
            inter, union = pair(g)
            w = inter * bu > bi * union
            bi = jnp.where(w, inter, bi)
            bu = jnp.where(w, union, bu)
            bidx = jnp.where(w, g.astype(jnp.float32), bidx)
            return bi, bu, bidx

        bi, bu, bidx = jax.lax.fori_loop(1, n_gt, mstep, (bi, bu, bidx), unroll=7)

        m = bi * 2.0 >= bu
        mf = m.astype(jnp.float32)

        c0 = cls_ref[i, 0]
        c1 = cls_ref[i, 1]
        mx = jnp.maximum(c0, c1)
        lse = mx + jnp.log(jnp.exp(c0 - mx) + jnp.exp(c1 - mx))
        ce = lse - jnp.where(m, c1, c0)

        tab_i = tab_ref[i]
        bidx_i = bidx.astype(jnp.int16)
        acc_reg = jnp.zeros((1, LANES), jnp.float32)
        acc_ldm = jnp.zeros((1, LANES), jnp.float32)
        for r in range(RB):
            ohr = jnp.where(iota == bidx_i[r:r + 1, :], onec, zeroc)
            t_r = jnp.dot(tab_i, ohr, preferred_element_type=jnp.float32)
            d = pq_ref[i, r] - t_r
            e0 = d[12:13, :] * iaw[r:r + 1, :]
            e1 = d[13:14, :] * iah[r:r + 1, :]
            sl = _smooth_l1(jnp.concatenate([d[0:12, :], e0, e1], axis=0))
            sl = sl * mf[r:r + 1, :]
            acc_ldm = acc_ldm + jnp.sum(sl[0:10, :], axis=0, keepdims=True)
            acc_reg = acc_reg + jnp.sum(sl[10:14, :], axis=0, keepdims=True)

        out_ref[i, 0] = out_ref[i, 0] + ce
        out_ref[i, 3] = out_ref[i, 3] + mf
        out_ref[i, 1, 0:1, :] = out_ref[i, 1, 0:1, :] + acc_reg
        out_ref[i, 2, 0:1, :] = out_ref[i, 2, 0:1, :] + acc_ldm


def kernel(pred_cls, pred_bbox, pred_landmarks, anchors, gt_boxes, gt_landmarks):
    n, a, g = pred_cls.shape[0], pred_cls.shape[1], gt_boxes.shape[1]
    ra = a // LANES
    grid = ra // RB

    ancT = anchors.T.reshape(4, ra, LANES)
    clsT = pred_cls.transpose(0, 2, 1).reshape(n, 2, ra, LANES)

    aw = anchors[:, 2] - anchors[:, 0]
    ah = anchors[:, 3] - anchors[:, 1]
    acx = (anchors[:, 0] + anchors[:, 2]) * 0.5
    acy = (anchors[:, 1] + anchors[:, 3]) * 0.5
    law = jnp.log(aw)
    lah = jnp.log(ah)

    pq = jnp.concatenate([
        pred_landmarks,
        (pred_bbox[..., 2] + law)[..., None],
        (pred_bbox[..., 3] + lah)[..., None],
        (pred_bbox[..., 0] * aw + acx)[..., None],
        (pred_bbox[..., 1] * ah + acy)[..., None],
        jnp.zeros((n, a, 2), jnp.float32)], axis=-1)
    pq = pq.reshape(n, ra, LANES, 16).transpose(0, 1, 3, 2)

    gw = gt_boxes[..., 2] - gt_boxes[..., 0]
    gh = gt_boxes[..., 3] - gt_boxes[..., 1]
    tabc = jnp.concatenate([
        gt_landmarks,
        jnp.log(gw)[..., None],
        jnp.log(gh)[..., None],
        ((gt_boxes[..., 0] + gt_boxes[..., 2]) * 0.5)[..., None],
        ((gt_boxes[..., 1] + gt_boxes[..., 3]) * 0.5)[..., None],
        jnp.zeros((n, g, 2), jnp.float32)], axis=-1)
    tab = jnp.zeros((n, 16, LANES), jnp.bfloat16)
    tab = tab.at[:, :, :g].set(tabc.transpose(0, 2, 1).astype(jnp.bfloat16))

    gtab = jnp.concatenate([gt_boxes, (gw * gh)[..., None]], axis=-1).reshape(n * g, 5)

    out = pl.pallas_call(
        functools.partial(_body, n, g),
        grid=(grid,),
        in_specs=[
            pl.BlockSpec(memory_space=pltpu.SMEM),
            pl.BlockSpec((n, 16, LANES), lambda j: (0, 0, 0)),
            pl.BlockSpec((4, RB, LANES), lambda j: (0, j, 0)),
            pl.BlockSpec((n, 2, RB, LANES), lambda j: (0, 0, j, 0)),
            pl.BlockSpec((n, RB, 16, LANES), lambda j: (0, j, 0, 0)),
        ],
        out_specs=pl.BlockSpec((n, 4, RB, LANES), lambda j: (0, 0, 0, 0)),
        out_shape=jax.ShapeDtypeStruct((n, 4, RB, LANES), jnp.float32),
    )(gtab, tab, ancT, clsT, pq)

    sums = out.sum(axis=(2, 3))
    npos = sums[:, 3]
    cls_loss = jnp.mean(sums[:, 0] / a)
    reg_loss = jnp.mean(sums[:, 1] / (npos * 4.0))
    ldm_loss = jnp.mean(sums[:, 2] / (npos * 10.0))
    total = cls_loss + reg_loss + ldm_loss
    return (total, cls_loss, reg_loss, ldm_loss)


# speedup vs baseline: 2.0916x; 2.0916x over previous
import jax
import jax.numpy as jnp
from jax.experimental import pallas as pl
from jax.experimental.pallas import tpu as pltpu

LANES = 128
RB = 8


def _smooth_l1(d):
    a = jnp.abs(d)
    return jnp.where(a < 1.0, 0.5 * d * d, a - 0.5)


def _body(n_batch, n_gt, gtab_ref, gtv_ref, anc_ref, cls_ref, bbox_ref, ldm_ref, out_ref):
    j = pl.program_id(0)

    @pl.when(j == 0)
    def _():
        out_ref[...] = jnp.zeros_like(out_ref)

    x0 = anc_ref[0]
    y0 = anc_ref[1]
    x1 = anc_ref[2]
    y1 = anc_ref[3]
    aw = x1 - x0
    ah = y1 - y0
    a1 = aw * ah
    acx = (x0 + x1) * 0.5
    acy = (y0 + y1) * 0.5
    iaw = 1.0 / aw
    iah = 1.0 / ah
    law = jnp.log(aw)
    lah = jnp.log(ah)

    def batch_body(i, carry):
        base = i * n_gt

        def pair(g):
            ixmin = jnp.maximum(x0, gtab_ref[base + g, 0])
            iymin = jnp.maximum(y0, gtab_ref[base + g, 1])
            ixmax = jnp.minimum(x1, gtab_ref[base + g, 2])
            iymax = jnp.minimum(y1, gtab_ref[base + g, 3])
            dx = jnp.maximum(ixmax - ixmin, 0.0)
            dy = jnp.maximum(iymax - iymin, 0.0)
            inter = dx * dy
            union = (a1 + gtab_ref[base + g, 4]) - inter
            return inter, union

        def trow(g, c):
            return gtv_ref[base + g, c]

        bi, bu = pair(0)
        tvp = [trow(0, c) for c in range(7)]

        for g in range(1, n_gt):
            inter, union = pair(g)
            w = inter * bu > bi * union
            bi = jnp.where(w, inter, bi)
            bu = jnp.where(w, union, bu)
            tvp = [jnp.where(w, trow(g, c), t) for c, t in enumerate(tvp)]

        tv = []
        for t in tvp:
            u = jax.lax.bitcast_convert_type(t, jnp.uint32)
            tv.append(jax.lax.bitcast_convert_type(u & jnp.uint32(0xFFFF0000), jnp.float32))
            tv.append(jax.lax.bitcast_convert_type(u << 16, jnp.float32))

        m = bi * 2.0 >= bu
        mf = m.astype(jnp.float32)

        c0 = cls_ref[i, 0]
        c1 = cls_ref[i, 1]
        mx = jnp.maximum(c0, c1)
        lse = mx + jnp.log(jnp.exp(c0 - mx) + jnp.exp(c1 - mx))
        ce = lse - jnp.where(m, c1, c0)

        rt0 = (tv[0] - acx) * iaw
        rt1 = (tv[1] - acy) * iah
        rt2 = tv[2] - law
        rt3 = tv[3] - lah
        regs = (_smooth_l1(bbox_ref[i, 0] - rt0) + _smooth_l1(bbox_ref[i, 1] - rt1)
                + _smooth_l1(bbox_ref[i, 2] - rt2) + _smooth_l1(bbox_ref[i, 3] - rt3))
        regs = regs * mf

        ldms = _smooth_l1(ldm_ref[i, 0] - tv[4])
        for c in range(1, 10):
            ldms = ldms + _smooth_l1(ldm_ref[i, c] - tv[4 + c])
        ldms = ldms * mf

        out_ref[i, 0] = out_ref[i, 0] + ce
        out_ref[i, 1] = out_ref[i, 1] + regs
        out_ref[i, 2] = out_ref[i, 2] + ldms
        out_ref[i, 3] = out_ref[i, 3] + mf
        return carry

    jax.lax.fori_loop(0, n_batch, batch_body, 0)


def kernel(pred_cls, pred_bbox, pred_landmarks, anchors, gt_boxes, gt_landmarks):
    n, a, g = pred_cls.shape[0], pred_cls.shape[1], gt_boxes.shape[1]
    ra = a // LANES
    grid = ra // RB

    ancT = anchors.T.reshape(4, ra, LANES)
    clsT = pred_cls.transpose(0, 2, 1).reshape(n, 2, ra, LANES)
    bboxT = pred_bbox.transpose(0, 2, 1).reshape(n, 4, ra, LANES)
    ldmT = pred_landmarks.transpose(0, 2, 1).reshape(n, 10, ra, LANES)

    gw = gt_boxes[..., 2] - gt_boxes[..., 0]
    gh = gt_boxes[..., 3] - gt_boxes[..., 1]
    gtab = jnp.concatenate(
        [gt_boxes,
         (gw * gh)[..., None],
         ((gt_boxes[..., 0] + gt_boxes[..., 2]) * 0.5)[..., None],
         ((gt_boxes[..., 1] + gt_boxes[..., 3]) * 0.5)[..., None],
         jnp.log(gw)[..., None],
         jnp.log(gh)[..., None],
         gt_landmarks], axis=-1).reshape(n * g, 19)
    tgt = gtab[:, 5:19]
    hi = jax.lax.bitcast_convert_type(tgt[:, 0::2].astype(jnp.bfloat16), jnp.uint16).astype(jnp.uint32)
    lo = jax.lax.bitcast_convert_type(tgt[:, 1::2].astype(jnp.bfloat16), jnp.uint16).astype(jnp.uint32)
    packed = jax.lax.bitcast_convert_type((hi << 16) | lo, jnp.float32)
    gtv = jnp.broadcast_to(packed[:, :, None, None], (n * g, 7, RB, LANES))

    import functools
    out = pl.pallas_call(
        functools.partial(_body, n, g),
        grid=(grid,),
        in_specs=[
            pl.BlockSpec(memory_space=pltpu.SMEM),
            pl.BlockSpec((n * g, 7, RB, LANES), lambda j: (0, 0, 0, 0)),
            pl.BlockSpec((4, RB, LANES), lambda j: (0, j, 0)),
            pl.BlockSpec((n, 2, RB, LANES), lambda j: (0, 0, j, 0)),
            pl.BlockSpec((n, 4, RB, LANES), lambda j: (0, 0, j, 0)),
            pl.BlockSpec((n, 10, RB, LANES), lambda j: (0, 0, j, 0)),
        ],
        out_specs=pl.BlockSpec((n, 4, RB, LANES), lambda j: (0, 0, 0, 0)),
        out_shape=jax.ShapeDtypeStruct((n, 4, RB, LANES), jnp.float32),
    )(gtab, gtv, ancT, clsT, bboxT, ldmT)

    sums = out.sum(axis=(2, 3))
    npos = sums[:, 3]
    cls_loss = jnp.mean(sums[:, 0] / a)
    reg_loss = jnp.mean(sums[:, 1] / (npos * 4.0))
    ldm_loss = jnp.mean(sums[:, 2] / (npos * 10.0))
    total = cls_loss + reg_loss + ldm_loss
    return (total, cls_loss, reg_loss, ldm_loss)


# speedup vs baseline: 2.3423x; 1.1199x over previous
import jax
import jax.numpy as jnp
from jax.experimental import pallas as pl
from jax.experimental.pallas import tpu as pltpu

LANES = 128
RB = 8


def _smooth_l1(d):
    a = jnp.abs(d)
    return jnp.where(a < 1.0, 0.5 * d * d, a - 0.5)


def _body(n_batch, n_gt, gtv_ref, anc_ref, cls_ref, bbox_ref, ldm_ref, out_ref):
    j = pl.program_id(0)

    @pl.when(j == 0)
    def _():
        out_ref[...] = jnp.zeros_like(out_ref)

    x0 = anc_ref[0]
    y0 = anc_ref[1]
    x1 = anc_ref[2]
    y1 = anc_ref[3]
    aw = x1 - x0
    ah = y1 - y0
    a1 = aw * ah
    acx = (x0 + x1) * 0.5
    acy = (y0 + y1) * 0.5
    iaw = 1.0 / aw
    iah = 1.0 / ah
    law = jnp.log(aw)
    lah = jnp.log(ah)

    def batch_body(i, carry):
        base = i * n_gt

        def pair(g):
            ixmin = jnp.maximum(x0, gtv_ref[base + g, 0])
            iymin = jnp.maximum(y0, gtv_ref[base + g, 1])
            ixmax = jnp.minimum(x1, gtv_ref[base + g, 2])
            iymax = jnp.minimum(y1, gtv_ref[base + g, 3])
            dx = jnp.maximum(ixmax - ixmin, 0.0)
            dy = jnp.maximum(iymax - iymin, 0.0)
            inter = dx * dy
            union = (a1 + gtv_ref[base + g, 4]) - inter
            return inter, union

        def trow(g, c):
            return gtv_ref[base + g, 5 + c]

        bi, bu = pair(0)
        tvp = [trow(0, c) for c in range(7)]

        for g in range(1, n_gt):
            inter, union = pair(g)
            w = inter * bu > bi * union
            bi = jnp.where(w, inter, bi)
            bu = jnp.where(w, union, bu)
            tvp = [jnp.where(w, trow(g, c), t) for c, t in enumerate(tvp)]

        tv = []
        for t in tvp:
            u = jax.lax.bitcast_convert_type(t, jnp.uint32)
            tv.append(jax.lax.bitcast_convert_type(u & jnp.uint32(0xFFFF0000), jnp.float32))
            tv.append(jax.lax.bitcast_convert_type(u << 16, jnp.float32))

        m = bi * 2.0 >= bu
        mf = m.astype(jnp.float32)

        c0 = cls_ref[i, 0]
        c1 = cls_ref[i, 1]
        mx = jnp.maximum(c0, c1)
        lse = mx + jnp.log(jnp.exp(c0 - mx) + jnp.exp(c1 - mx))
        ce = lse - jnp.where(m, c1, c0)

        rt0 = (tv[0] - acx) * iaw
        rt1 = (tv[1] - acy) * iah
        rt2 = tv[2] - law
        rt3 = tv[3] - lah
        regs = (_smooth_l1(bbox_ref[i, 0] - rt0) + _smooth_l1(bbox_ref[i, 1] - rt1)
                + _smooth_l1(bbox_ref[i, 2] - rt2) + _smooth_l1(bbox_ref[i, 3] - rt3))
        regs = regs * mf

        ldms = _smooth_l1(ldm_ref[i, 0] - tv[4])
        for c in range(1, 10):
            ldms = ldms + _smooth_l1(ldm_ref[i, c] - tv[4 + c])
        ldms = ldms * mf

        out_ref[i, 0] = out_ref[i, 0] + ce
        out_ref[i, 1] = out_ref[i, 1] + regs
        out_ref[i, 2] = out_ref[i, 2] + ldms
        out_ref[i, 3] = out_ref[i, 3] + mf
        return carry

    jax.lax.fori_loop(0, n_batch, batch_body, 0, unroll=2)


def kernel(pred_cls, pred_bbox, pred_landmarks, anchors, gt_boxes, gt_landmarks):
    n, a, g = pred_cls.shape[0], pred_cls.shape[1], gt_boxes.shape[1]
    ra = a // LANES
    grid = ra // RB

    ancT = anchors.T.reshape(4, ra, LANES)
    clsT = pred_cls.transpose(0, 2, 1).reshape(n, 2, ra, LANES)
    bboxT = pred_bbox.transpose(0, 2, 1).reshape(n, 4, ra, LANES)
    ldmT = pred_landmarks.transpose(0, 2, 1).reshape(n, 10, ra, LANES)

    gw = gt_boxes[..., 2] - gt_boxes[..., 0]
    gh = gt_boxes[..., 3] - gt_boxes[..., 1]
    gtab = jnp.concatenate(
        [gt_boxes,
         (gw * gh)[..., None],
         ((gt_boxes[..., 0] + gt_boxes[..., 2]) * 0.5)[..., None],
         ((gt_boxes[..., 1] + gt_boxes[..., 3]) * 0.5)[..., None],
         jnp.log(gw)[..., None],
         jnp.log(gh)[..., None],
         gt_landmarks], axis=-1).reshape(n * g, 19)
    tgt = gtab[:, 5:19]
    hi = jax.lax.bitcast_convert_type(tgt[:, 0::2].astype(jnp.bfloat16), jnp.uint16).astype(jnp.uint32)
    lo = jax.lax.bitcast_convert_type(tgt[:, 1::2].astype(jnp.bfloat16), jnp.uint16).astype(jnp.uint32)
    packed = jax.lax.bitcast_convert_type((hi << 16) | lo, jnp.float32)
    rows = jnp.concatenate([gtab[:, :5], packed], axis=1)
    gtv = jnp.broadcast_to(rows[:, :, None, None], (n * g, 12, RB, LANES))

    import functools
    out = pl.pallas_call(
        functools.partial(_body, n, g),
        grid=(grid,),
        in_specs=[
            pl.BlockSpec((n * g, 12, RB, LANES), lambda j: (0, 0, 0, 0)),
            pl.BlockSpec((4, RB, LANES), lambda j: (0, j, 0)),
            pl.BlockSpec((n, 2, RB, LANES), lambda j: (0, 0, j, 0)),
            pl.BlockSpec((n, 4, RB, LANES), lambda j: (0, 0, j, 0)),
            pl.BlockSpec((n, 10, RB, LANES), lambda j: (0, 0, j, 0)),
        ],
        out_specs=pl.BlockSpec((n, 4, RB, LANES), lambda j: (0, 0, 0, 0)),
        out_shape=jax.ShapeDtypeStruct((n, 4, RB, LANES), jnp.float32),
    )(gtv, ancT, clsT, bboxT, ldmT)

    sums = out.sum(axis=(2, 3))
    npos = sums[:, 3]
    cls_loss = jnp.mean(sums[:, 0] / a)
    reg_loss = jnp.mean(sums[:, 1] / (npos * 4.0))
    ldm_loss = jnp.mean(sums[:, 2] / (npos * 10.0))
    total = cls_loss + reg_loss + ldm_loss
    return (total, cls_loss, reg_loss, ldm_loss)
